# Initial kernel scaffold; baseline (speedup 1.0000x reference)
import jax
import jax.numpy as jnp
from jax.experimental import pallas as pl


def kernel(input_ids, W_emb, W_proj, b_proj):
    raise NotImplementedError("write your pallas kernel here")


# speedup vs baseline: 2.5773x; 2.5773x over previous
import functools

import jax
import jax.numpy as jnp
from jax import lax
from jax.experimental import pallas as pl
from jax.experimental.pallas import tpu as pltpu
from jax.experimental.pallas import tpu_sc as plsc

_NW = 32
_D = 8

_GATHER_DNUMS = lax.GatherDimensionNumbers(
    offset_dims=(), collapsed_slice_dims=(0,), start_index_map=(0,)
)


def _vgather(vec, idx):
    return lax.gather(
        vec, idx[:, None], _GATHER_DNUMS, (1,),
        mode=lax.GatherScatterMode.PROMISE_IN_BOUNDS,
    )


@functools.lru_cache(maxsize=None)
def _gather_call(n_tok):
    tpw = n_tok // _NW
    opw = tpw * _D
    n_vec = opw // 16
    mesh = plsc.VectorSubcoreMesh(core_axis_name="c", subcore_axis_name="s")

    @functools.partial(
        pl.kernel,
        mesh=mesh,
        out_type=jax.ShapeDtypeStruct((n_tok * _D,), jnp.float32),
        scratch_types=[
            pltpu.VMEM((_D, 16), jnp.float32),
            pltpu.VMEM((tpw,), jnp.int32),
            pltpu.VMEM((opw,), jnp.float32),
        ],
    )
    def k(ids_hbm, t2_hbm, out_hbm, t2_v, ids_v, out_v):
        wid = lax.axis_index("s") * 2 + lax.axis_index("c")
        pltpu.sync_copy(t2_hbm, t2_v)
        pltpu.sync_copy(ids_hbm.at[pl.ds(wid * tpw, tpw)], ids_v)

        leaves = [t2_v[r, :] for r in range(_D)]
        half = lax.shift_right_logical(lax.iota(jnp.int32, 16), 3)
        pats = [2 * v + half for v in range(8)]

        def body(c, carry):
            idv = ids_v[pl.ds(c * 16, 16)]
            for v in range(8):
                ide = _vgather(idv, pats[v])
                b0 = lax.bitwise_and(ide, jnp.int32(1)) > 0
                b1 = lax.bitwise_and(ide, jnp.int32(2)) > 0
                b2 = ide >= 4
                s00 = jnp.where(b0, leaves[1], leaves[0])
                s01 = jnp.where(b0, leaves[3], leaves[2])
                s02 = jnp.where(b0, leaves[5], leaves[4])
                s03 = jnp.where(b0, leaves[7], leaves[6])
                s10 = jnp.where(b1, s01, s00)
                s11 = jnp.where(b1, s03, s02)
                out_v[pl.ds((c * 8 + v) * 16, 16)] = jnp.where(b2, s11, s10)
            return carry

        lax.fori_loop(0, n_vec // 8, body, 0)
        pltpu.sync_copy(out_v, out_hbm.at[pl.ds(wid * opw, opw)])

    return k


def _table_body(w_emb_ref, w_proj_ref, b_ref, t2_ref):
    t = lax.dot_general(
        w_emb_ref[...], w_proj_ref[...],
        (((1,), (1,)), ((), ())),
        preferred_element_type=jnp.float32,
    ) + b_ref[...]
    t2_ref[...] = jnp.concatenate([t, t], axis=1)


def _fused_table2(W_emb, W_proj, b_proj):
    return pl.pallas_call(
        _table_body,
        out_shape=jax.ShapeDtypeStruct((_D, 16), jnp.float32),
    )(W_emb, W_proj, b_proj.reshape(1, _D))


def kernel(input_ids, W_emb, W_proj, b_proj):
    b, s = input_ids.shape
    n_tok = b * s
    ids = input_ids.reshape(n_tok).astype(jnp.int32)
    t2 = _fused_table2(W_emb, W_proj, b_proj)
    out = _gather_call(n_tok)(ids, t2)
    return out.reshape(b, s, _D)


# speedup vs baseline: 3.1098x; 1.2066x over previous
import functools

import jax
import jax.numpy as jnp
from jax import lax
from jax.experimental import pallas as pl
from jax.experimental.pallas import tpu as pltpu
from jax.experimental.pallas import tpu_sc as plsc

_NW = 32
_D = 8
_CHUNK = 256

_GATHER_DNUMS = lax.GatherDimensionNumbers(
    offset_dims=(), collapsed_slice_dims=(0,), start_index_map=(0,)
)


def _vgather(vec, idx):
    return lax.gather(
        vec, idx[:, None], _GATHER_DNUMS, (1,),
        mode=lax.GatherScatterMode.PROMISE_IN_BOUNDS,
    )


@functools.lru_cache(maxsize=None)
def _gather_call(nb, ns):
    n_tok = nb * ns
    tpw = n_tok // _NW
    tiles_per_b = ns // tpw
    n_chunk = tpw // _CHUNK
    mesh = plsc.VectorSubcoreMesh(core_axis_name="c", subcore_axis_name="s")

    @functools.partial(
        pl.kernel,
        mesh=mesh,
        out_type=jax.ShapeDtypeStruct((nb, ns, 128), jnp.float32),
        scratch_types=[
            pltpu.VMEM((_D, 16), jnp.float32),
            pltpu.VMEM((tpw,), jnp.int32),
            pltpu.VMEM((_CHUNK, 128), jnp.float32),
        ],
    )
    def k(ids_hbm, t2_hbm, out_hbm, t2_v, ids_v, out_v):
        wid = lax.axis_index("s") * 2 + lax.axis_index("c")
        bidx = wid // tiles_per_b
        s0 = (wid % tiles_per_b) * tpw
        pltpu.sync_copy(t2_hbm, t2_v)
        pltpu.sync_copy(ids_hbm.at[bidx, pl.ds(s0, tpw)], ids_v)

        leaves = [t2_v[r, :] for r in range(_D)]
        lane = lax.iota(jnp.int32, 16)
        half = lax.shift_right_logical(lane, 3)
        rot8 = lax.bitwise_and(lane + 8, jnp.int32(15))

        def chunk_body(cc, carry):
            def group_body(g, carry2):
                idv = ids_v[pl.ds(cc * _CHUNK + g * 16, 16)]
                for v in range(8):
                    ide = _vgather(idv, 2 * v + half)
                    b0 = lax.bitwise_and(ide, jnp.int32(1)) > 0
                    b1 = lax.bitwise_and(ide, jnp.int32(2)) > 0
                    b2 = ide >= 4
                    s00 = jnp.where(b0, leaves[1], leaves[0])
                    s01 = jnp.where(b0, leaves[3], leaves[2])
                    s02 = jnp.where(b0, leaves[5], leaves[4])
                    s03 = jnp.where(b0, leaves[7], leaves[6])
                    s10 = jnp.where(b1, s01, s00)
                    s11 = jnp.where(b1, s03, s02)
                    pair = jnp.where(b2, s11, s10)
                    r = g * 16 + 2 * v
                    out_v[r, pl.ds(0, 16)] = pair
                    out_v[r + 1, pl.ds(0, 16)] = _vgather(pair, rot8)
                return carry2

            lax.fori_loop(0, _CHUNK // 16, group_body, 0)
            pltpu.sync_copy(
                out_v,
                out_hbm.at[bidx, pl.ds(s0 + cc * _CHUNK, _CHUNK), :],
            )
            return carry

        lax.fori_loop(0, n_chunk, chunk_body, 0)

    return k


def _table_body(w_emb_ref, w_proj_ref, b_ref, t2_ref):
    t = lax.dot_general(
        w_emb_ref[...], w_proj_ref[...],
        (((1,), (1,)), ((), ())),
        preferred_element_type=jnp.float32,
    ) + b_ref[...]
    t2_ref[...] = jnp.concatenate([t, t], axis=1)


def _fused_table2(W_emb, W_proj, b_proj):
    return pl.pallas_call(
        _table_body,
        out_shape=jax.ShapeDtypeStruct((_D, 16), jnp.float32),
    )(W_emb, W_proj, b_proj.reshape(1, _D))


def kernel(input_ids, W_emb, W_proj, b_proj):
    nb, ns = input_ids.shape
    ids = input_ids.astype(jnp.int32)
    t2 = _fused_table2(W_emb, W_proj, b_proj)
    out128 = _gather_call(nb, ns)(ids, t2)
    return lax.slice(out128, (0, 0, 0), (nb, ns, _D))


# speedup vs baseline: 3.1815x; 1.0231x over previous
import functools

import jax
import jax.numpy as jnp
from jax import lax
from jax.experimental import pallas as pl
from jax.experimental.pallas import tpu as pltpu
from jax.experimental.pallas import tpu_sc as plsc

_NW = 32
_D = 8
_CHUNK = 256

_GATHER_DNUMS = lax.GatherDimensionNumbers(
    offset_dims=(), collapsed_slice_dims=(0,), start_index_map=(0,)
)


def _vgather(vec, idx):
    return lax.gather(
        vec, idx[:, None], _GATHER_DNUMS, (1,),
        mode=lax.GatherScatterMode.PROMISE_IN_BOUNDS,
    )


@functools.lru_cache(maxsize=None)
def _gather_call(nb, ns):
    n_tok = nb * ns
    tpw = n_tok // _NW
    tiles_per_b = ns // tpw
    n_chunk = tpw // _CHUNK
    mesh = plsc.VectorSubcoreMesh(core_axis_name="c", subcore_axis_name="s")

    @functools.partial(
        pl.kernel,
        mesh=mesh,
        out_type=jax.ShapeDtypeStruct((nb, ns, 128), jnp.float32),
        scratch_types=[
            pltpu.VMEM((_D, 16), jnp.float32),
            pltpu.VMEM((tpw,), jnp.int32),
            pltpu.VMEM((_CHUNK, 128), jnp.float32),
            pltpu.VMEM((_CHUNK, 128), jnp.float32),
            pltpu.SemaphoreType.DMA,
            pltpu.SemaphoreType.DMA,
        ],
    )
    def k(ids_hbm, t2_hbm, out_hbm, t2_v, ids_v, out_a, out_b, sem_a, sem_b):
        wid = lax.axis_index("s") * 2 + lax.axis_index("c")
        bidx = wid // tiles_per_b
        s0 = (wid % tiles_per_b) * tpw
        pltpu.sync_copy(t2_hbm, t2_v)
        pltpu.sync_copy(ids_hbm.at[bidx, pl.ds(s0, tpw)], ids_v)

        leaves = [t2_v[r, :] for r in range(_D)]
        lane = lax.iota(jnp.int32, 16)
        half = lax.shift_right_logical(lane, 3)
        rot8 = lax.bitwise_and(lane + 8, jnp.int32(15))

        bufs = (out_a, out_b)
        sems = (sem_a, sem_b)
        handles = [None, None]
        for cc in range(n_chunk):
            p = cc & 1
            if handles[p] is not None:
                handles[p].wait()
            out_v = bufs[p]

            def group_body(g, carry2, cc=cc, out_v=out_v):
                idv = ids_v[pl.ds(cc * _CHUNK + g * 16, 16)]
                for v in range(8):
                    ide = _vgather(idv, 2 * v + half)
                    b0 = lax.bitwise_and(ide, jnp.int32(1)) > 0
                    b1 = lax.bitwise_and(ide, jnp.int32(2)) > 0
                    b2 = ide >= 4
                    s00 = jnp.where(b0, leaves[1], leaves[0])
                    s01 = jnp.where(b0, leaves[3], leaves[2])
                    s02 = jnp.where(b0, leaves[5], leaves[4])
                    s03 = jnp.where(b0, leaves[7], leaves[6])
                    s10 = jnp.where(b1, s01, s00)
                    s11 = jnp.where(b1, s03, s02)
                    pair = jnp.where(b2, s11, s10)
                    r = g * 16 + 2 * v
                    out_v[r, pl.ds(0, 16)] = pair
                    out_v[r + 1, pl.ds(0, 16)] = _vgather(pair, rot8)
                return carry2

            lax.fori_loop(0, _CHUNK // 16, group_body, 0)
            handles[p] = pltpu.async_copy(
                out_v,
                out_hbm.at[bidx, pl.ds(s0 + cc * _CHUNK, _CHUNK), :],
                sems[p],
            )
        for h in handles:
            if h is not None:
                h.wait()

    return k


def _table_body(w_emb_ref, w_proj_ref, b_ref, t2_ref):
    t = lax.dot_general(
        w_emb_ref[...], w_proj_ref[...],
        (((1,), (1,)), ((), ())),
        preferred_element_type=jnp.float32,
    ) + b_ref[...]
    t2_ref[...] = jnp.concatenate([t, t], axis=1)


def _fused_table2(W_emb, W_proj, b_proj):
    return pl.pallas_call(
        _table_body,
        out_shape=jax.ShapeDtypeStruct((_D, 16), jnp.float32),
    )(W_emb, W_proj, b_proj.reshape(1, _D))


def kernel(input_ids, W_emb, W_proj, b_proj):
    nb, ns = input_ids.shape
    ids = input_ids.astype(jnp.int32)
    t2 = _fused_table2(W_emb, W_proj, b_proj)
    out128 = _gather_call(nb, ns)(ids, t2)
    return lax.slice(out128, (0, 0, 0), (nb, ns, _D))


# speedup vs baseline: 3.2133x; 1.0100x over previous
import functools

import jax
import jax.numpy as jnp
from jax import lax
from jax.experimental import pallas as pl
from jax.experimental.pallas import tpu as pltpu
from jax.experimental.pallas import tpu_sc as plsc

_NW = 32
_D = 8
_CHUNK = 256

_GATHER_DNUMS = lax.GatherDimensionNumbers(
    offset_dims=(), collapsed_slice_dims=(0,), start_index_map=(0,)
)


def _vgather(vec, idx):
    return lax.gather(
        vec, idx[:, None], _GATHER_DNUMS, (1,),
        mode=lax.GatherScatterMode.PROMISE_IN_BOUNDS,
    )


@functools.lru_cache(maxsize=None)
def _gather_call(nb, ns):
    n_tok = nb * ns
    tpw = n_tok // _NW
    tiles_per_b = ns // tpw
    n_chunk = tpw // _CHUNK
    mesh = plsc.VectorSubcoreMesh(core_axis_name="c", subcore_axis_name="s")

    @functools.partial(
        pl.kernel,
        mesh=mesh,
        out_type=jax.ShapeDtypeStruct((nb, ns, 128), jnp.float32),
        scratch_types=[
            pltpu.VMEM((_D, 16), jnp.float32),
            pltpu.VMEM((tpw,), jnp.int32),
            pltpu.VMEM((_CHUNK, 128), jnp.float32),
            pltpu.VMEM((_CHUNK, 128), jnp.float32),
            pltpu.SemaphoreType.DMA,
            pltpu.SemaphoreType.DMA,
        ],
    )
    def k(ids_hbm, t2_hbm, out_hbm, t2_v, ids_v, out_a, out_b, sem_a, sem_b):
        wid = lax.axis_index("s") * 2 + lax.axis_index("c")
        bidx = wid // tiles_per_b
        s0 = (wid % tiles_per_b) * tpw
        pltpu.sync_copy(t2_hbm, t2_v)
        pltpu.sync_copy(ids_hbm.at[bidx, pl.ds(s0, tpw)], ids_v)

        leaves = [t2_v[r, :] for r in range(_D)]
        lane = lax.iota(jnp.int32, 16)
        half = lax.shift_right_logical(lane, 3)
        rot8 = lax.bitwise_and(lane + 8, jnp.int32(15))

        bufs = (out_a, out_b)
        sems = (sem_a, sem_b)
        handles = [None, None]
        for cc in range(n_chunk):
            p = cc & 1
            if handles[p] is not None:
                handles[p].wait()
            out_v = bufs[p]

            def group_body(g, carry2, cc=cc, out_v=out_v):
                idv = ids_v[pl.ds(cc * _CHUNK + g * 16, 16)]

                def pair_body(v, carry3, out_v=out_v):
                    ide = _vgather(idv, 2 * v + half)
                    b0 = lax.bitwise_and(ide, jnp.int32(1)) > 0
                    b1 = lax.bitwise_and(ide, jnp.int32(2)) > 0
                    b2 = ide >= 4
                    s00 = jnp.where(b0, leaves[1], leaves[0])
                    s01 = jnp.where(b0, leaves[3], leaves[2])
                    s02 = jnp.where(b0, leaves[5], leaves[4])
                    s03 = jnp.where(b0, leaves[7], leaves[6])
                    s10 = jnp.where(b1, s01, s00)
                    s11 = jnp.where(b1, s03, s02)
                    pair = jnp.where(b2, s11, s10)
                    r = g * 16 + 2 * v
                    out_v[r, pl.ds(0, 16)] = pair
                    out_v[r + 1, pl.ds(0, 16)] = _vgather(pair, rot8)
                    return carry3

                return lax.fori_loop(0, 8, pair_body, carry2)

            lax.fori_loop(0, _CHUNK // 16, group_body, 0)
            handles[p] = pltpu.async_copy(
                out_v,
                out_hbm.at[bidx, pl.ds(s0 + cc * _CHUNK, _CHUNK), :],
                sems[p],
            )
        for h in handles:
            if h is not None:
                h.wait()

    return k


def _table_body(w_emb_ref, w_proj_ref, b_ref, t2_ref):
    t = lax.dot_general(
        w_emb_ref[...], w_proj_ref[...],
        (((1,), (1,)), ((), ())),
        preferred_element_type=jnp.float32,
    ) + b_ref[...]
    t2_ref[...] = jnp.concatenate([t, t], axis=1)


def _fused_table2(W_emb, W_proj, b_proj):
    return pl.pallas_call(
        _table_body,
        out_shape=jax.ShapeDtypeStruct((_D, 16), jnp.float32),
    )(W_emb, W_proj, b_proj.reshape(1, _D))


def kernel(input_ids, W_emb, W_proj, b_proj):
    nb, ns = input_ids.shape
    ids = input_ids.astype(jnp.int32)
    t2 = _fused_table2(W_emb, W_proj, b_proj)
    out128 = _gather_call(nb, ns)(ids, t2)
    return lax.slice(out128, (0, 0, 0), (nb, ns, _D))
